# Initial kernel scaffold; baseline (speedup 1.0000x reference)
import jax
import jax.numpy as jnp
from jax.experimental import pallas as pl


def kernel(a_hat, x, w1, w2, wgx, wge, wm1, wm2, tw):
    raise NotImplementedError("write your pallas kernel here")


# speedup vs baseline: 1.2826x; 1.2826x over previous
import functools

import jax
import jax.numpy as jnp
from jax.experimental import pallas as pl
from jax.experimental.pallas import tpu as pltpu

_F32 = jnp.float32
_BF16 = jnp.bfloat16


def _rup(v, m):
    return ((v + m - 1) // m) * m


def _pad2(x, shape):
    pads = [(0, t - s) for s, t in zip(x.shape, shape)]
    if all(p == (0, 0) for p in pads):
        return x
    return jnp.pad(x, pads)


def _emb_pre_kernel(a_ref, x_ref, w1_ref, wgxc_ref, wec_ref, pre_ref, acc_ref,
                    *, tm, tk):
    k = pl.program_id(1)

    @pl.when(k == 0)
    def _():
        acc_ref[...] = jnp.zeros_like(acc_ref)

    xk = x_ref[pl.ds(k * tk, tk), :]
    xw = jnp.dot(xk, w1_ref[...], preferred_element_type=_F32)
    acc_ref[...] += jnp.dot(a_ref[...], xw, preferred_element_type=_F32)

    @pl.when(k == pl.num_programs(1) - 1)
    def _():
        i = pl.program_id(0)
        emb = jnp.maximum(acc_ref[...], 0.0).astype(_BF16)
        xi = x_ref[pl.ds(i * tm, tm), :]
        pre = (jnp.dot(xi, wgxc_ref[...], preferred_element_type=_F32)
               + jnp.dot(emb, wec_ref[...], preferred_element_type=_F32))
        pre_ref[...] = pre.astype(_BF16)


def _prop2_kernel(a_ref, pre_ref, x_ref, wm1_ref, wm2_ref, tw_ref,
                  base_ref, ex_ref, mlp_ref, t_ref, acc_ref, *, tk, c):
    k = pl.program_id(1)

    @pl.when(k == 0)
    def _():
        acc_ref[...] = jnp.zeros_like(acc_ref)

    prek = pre_ref[pl.ds(k * tk, tk), :]
    acc_ref[...] += jnp.dot(a_ref[...], prek, preferred_element_type=_F32)

    @pl.when(k == pl.num_programs(1) - 1)
    def _():
        mh = jnp.maximum(
            jnp.dot(x_ref[...], wm1_ref[...], preferred_element_type=_F32), 0.0)
        mlp = jnp.dot(mh.astype(_BF16), wm2_ref[...],
                      preferred_element_type=_F32)
        base = acc_ref[:, 0:c]
        ex = acc_ref[:, c:2 * c]
        base_ref[...] = base
        ex_ref[...] = ex
        mlp_ref[...] = mlp
        t_ref[...] = tw_ref[0] * base + tw_ref[1] * ex + tw_ref[2] * mlp


def kernel(a_hat, x, w1, w2, wgx, wge, wm1, wm2, tw):
    n, f = x.shape
    h = w1.shape[1]
    c = w2.shape[1]

    LANE = 128
    TILE = 1024

    n_p = _rup(n, LANE)
    if n_p > TILE:
        n_p = _rup(n, TILE)
        tm = tk = TILE
    else:
        tk = n_p
        tm = n_p // 2
    f_p = _rup(f, LANE)
    h_p = _rup(h, LANE)
    wpre = _rup(2 * c, LANE)

    grid_m, grid_k = n_p // tm, n_p // tk
    grid = (grid_m, grid_k)

    a_p = _pad2(a_hat.astype(_F32), (n_p, n_p))
    x_bf = _pad2(x.astype(_F32), (n_p, f_p)).astype(_BF16)

    w1_bf = _pad2(w1.astype(_F32), (f_p, h_p)).astype(_BF16)
    wgxc = jnp.zeros((f_p, wpre), _F32)
    wgxc = wgxc.at[:f, c:2 * c].set(wgx.astype(_F32))
    wgxc_bf = wgxc.astype(_BF16)
    wec = jnp.zeros((h_p, wpre), _F32)
    wec = wec.at[:h, 0:c].set(w2.astype(_F32))
    wec = wec.at[:h, c:2 * c].set(wge.astype(_F32))
    wec_bf = wec.astype(_BF16)
    wm1_bf = _pad2(wm1.astype(_F32), (f_p, h_p)).astype(_BF16)
    wm2_bf = _pad2(wm2.astype(_F32), (h_p, c)).astype(_BF16)
    tw_f = tw.astype(_F32)

    cparams = pltpu.CompilerParams(
        dimension_semantics=("parallel", "arbitrary"),
        vmem_limit_bytes=64 * 1024 * 1024)

    cost1 = pl.CostEstimate(
        flops=int(2 * n_p * n_p * h_p + 2 * n_p * f_p * h_p * grid_m),
        transcendentals=0,
        bytes_accessed=int(4 * n_p * n_p + 2 * n_p * f_p + 2 * n_p * wpre))

    pre = pl.pallas_call(
        functools.partial(_emb_pre_kernel, tm=tm, tk=tk),
        out_shape=jax.ShapeDtypeStruct((n_p, wpre), _BF16),
        grid=grid,
        in_specs=[
            pl.BlockSpec((tm, tk), lambda i, k: (i, k)),
            pl.BlockSpec((n_p, f_p), lambda i, k: (0, 0)),
            pl.BlockSpec((f_p, h_p), lambda i, k: (0, 0)),
            pl.BlockSpec((f_p, wpre), lambda i, k: (0, 0)),
            pl.BlockSpec((h_p, wpre), lambda i, k: (0, 0)),
        ],
        out_specs=pl.BlockSpec((tm, wpre), lambda i, k: (i, 0)),
        scratch_shapes=[pltpu.VMEM((tm, h_p), _F32)],
        compiler_params=cparams,
        cost_estimate=cost1,
    )(a_p, x_bf, w1_bf, wgxc_bf, wec_bf)

    cost2 = pl.CostEstimate(
        flops=int(2 * n_p * n_p * wpre
                  + 2 * n_p * f_p * h_p + 2 * n_p * h_p * c),
        transcendentals=0,
        bytes_accessed=int(4 * n_p * n_p + 2 * n_p * wpre + 2 * n_p * f_p
                           + 4 * 4 * n_p * c))

    outs = pl.pallas_call(
        functools.partial(_prop2_kernel, tk=tk, c=c),
        out_shape=tuple(jax.ShapeDtypeStruct((n_p, c), _F32)
                        for _ in range(4)),
        grid=grid,
        in_specs=[
            pl.BlockSpec((tm, tk), lambda i, k: (i, k)),
            pl.BlockSpec((n_p, wpre), lambda i, k: (0, 0)),
            pl.BlockSpec((tm, f_p), lambda i, k: (i, 0)),
            pl.BlockSpec((f_p, h_p), lambda i, k: (0, 0)),
            pl.BlockSpec((h_p, c), lambda i, k: (0, 0)),
            pl.BlockSpec(memory_space=pltpu.SMEM),
        ],
        out_specs=tuple(pl.BlockSpec((tm, c), lambda i, k: (i, 0))
                        for _ in range(4)),
        scratch_shapes=[pltpu.VMEM((tm, wpre), _F32)],
        compiler_params=cparams,
        cost_estimate=cost2,
    )(a_p, pre, x_bf, wm1_bf, wm2_bf, tw_f)

    base_p, ex_p, mlp_p, t_p = outs
    if n_p != n:
        base_p, ex_p, mlp_p, t_p = (
            base_p[:n], ex_p[:n], mlp_p[:n], t_p[:n])
    return base_p, ex_p, mlp_p, t_p
